# Initial kernel scaffold; baseline (speedup 1.0000x reference)
import jax
import jax.numpy as jnp
from jax.experimental import pallas as pl


def kernel(x_batch, y_batch, eps, n_projections):
    raise NotImplementedError("write your pallas kernel here")


# speedup vs baseline: 5.4636x; 5.4636x over previous
import functools

import jax
import jax.numpy as jnp
from jax import lax
from jax.experimental import pallas as pl
from jax.experimental.pallas import tpu as pltpu
from jax.experimental.pallas import tpu_sc as plsc

_B, _N, _D, _P = 4, 8192, 64, 16
_W = _B * _P


def _thetas():
    ths = []
    for i in range(_P):
        tk = jax.random.fold_in(jax.random.key(42), i)
        th = jax.random.normal(tk, (_D,), dtype=jnp.float32)
        th = th / jnp.maximum(jnp.sqrt(jnp.sum(th ** 2)), 1e-12)
        ths.append(th)
    return jnp.stack(ths)


def _cmpex_reshape(v, p, j, k):
    n, w = v.shape
    g = n // (2 * j)
    vv = v.reshape(g, 2, j, w)
    a_v, b_v = vv[:, 0], vv[:, 1]
    gi = lax.broadcasted_iota(jnp.int32, (g, 1, 1), 0)
    asc = ((gi * (2 * j)) & k) == 0
    if p is not None:
        pp = p.reshape(g, 2, j, w)
        a_p, b_p = pp[:, 0], pp[:, 1]
        a_small = (a_v < b_v) | ((a_v == b_v) & (a_p < b_p))
    else:
        a_small = a_v < b_v
    swap = jnp.logical_xor(asc, a_small)
    v = jnp.concatenate(
        [jnp.where(swap, b_v, a_v)[:, None],
         jnp.where(swap, a_v, b_v)[:, None]], axis=1).reshape(n, w)
    if p is None:
        return v, None
    p = jnp.concatenate(
        [jnp.where(swap, b_p, a_p)[:, None],
         jnp.where(swap, a_p, b_p)[:, None]], axis=1).reshape(n, w)
    return v, p


def _cmpex_small(v, p, j, k):
    n, w = v.shape
    g = n // 8
    v8 = v.reshape(g, 8, w)
    p8 = p.reshape(g, 8, w) if p is not None else None
    slab = lax.broadcasted_iota(jnp.int32, (g, 1, 1), 0)
    pieces_v, pieces_p = [], []
    for sub in range(0, 8, 2 * j):
        a_v = v8[:, sub:sub + j]
        b_v = v8[:, sub + j:sub + 2 * j]
        if p is not None:
            a_p = p8[:, sub:sub + j]
            b_p = p8[:, sub + j:sub + 2 * j]
            a_small = (a_v < b_v) | ((a_v == b_v) & (a_p < b_p))
        else:
            a_small = a_v < b_v
        if k < 8:
            swap = jnp.logical_not(a_small) if (sub & k) == 0 else a_small
        else:
            asc = ((slab * 8) & k) == 0
            swap = jnp.logical_xor(asc, a_small)
        pieces_v += [jnp.where(swap, b_v, a_v), jnp.where(swap, a_v, b_v)]
        if p is not None:
            pieces_p += [jnp.where(swap, b_p, a_p), jnp.where(swap, a_p, b_p)]
    v = jnp.concatenate(pieces_v, axis=1).reshape(n, w)
    if p is None:
        return v, None
    p = jnp.concatenate(pieces_p, axis=1).reshape(n, w)
    return v, p


def _bitonic(v, p):
    n = v.shape[0]
    k = 2
    while k <= n:
        j = k // 2
        while j >= 1:
            if j >= 8:
                v, p = _cmpex_reshape(v, p, j, k)
            else:
                v, p = _cmpex_small(v, p, j, k)
            j //= 2
        k *= 2
    return v, p


def _bitonic_ref(v_scr, p_scr):
    n = v_scr.shape[0]
    k = 2
    while k <= n:
        j = k // 2
        while j >= 1:
            v = v_scr[...]
            p = p_scr[...] if p_scr is not None else None
            if j >= 8:
                v, p = _cmpex_reshape(v, p, j, k)
            else:
                v, p = _cmpex_small(v, p, j, k)
            v_scr[...] = v
            if p_scr is not None:
                p_scr[...] = p
            j //= 2
        k *= 2


def _proj_body(x_ref, y_ref, th_ref, xp_ref, yp_ref):
    th = th_ref[...]
    xp_ref[0] = lax.dot_general(
        th, x_ref[0], (((1,), (1,)), ((), ())),
        preferred_element_type=jnp.float32)
    yp_ref[0] = lax.dot_general(
        th, y_ref[0], (((1,), (1,)), ((), ())),
        preferred_element_type=jnp.float32)


def _sort_body(xp_ref, yp_ref, idx_ref, d_ref, v_scr, p_scr):
    allp = jnp.concatenate(
        [xp_ref[b] for b in range(_B)] + [yp_ref[b] for b in range(_B)],
        axis=0)
    v_scr[...] = allp.T
    p_scr[...] = lax.broadcasted_iota(jnp.int32, (_N, 2 * _W), 0)
    _bitonic_ref(v_scr, p_scr)
    v = v_scr[...]
    idx_ref[...] = p_scr[:, :_W].T
    d_ref[...] = (v[:, _W:] - v[:, :_W]).T


def _final_body(x_ref, diff_ref, tho_ref, o_ref):
    ob = lax.dot_general(
        diff_ref[...], tho_ref[...], (((0,), (0,)), ((), ())),
        preferred_element_type=jnp.float32)
    o_ref[0] = x_ref[0] + ob


def _make_scatter():
    mesh = plsc.VectorSubcoreMesh(core_axis_name="c", subcore_axis_name="s")
    rows_per = _W // 32

    @functools.partial(
        pl.kernel, mesh=mesh,
        out_type=jax.ShapeDtypeStruct((_W, _N), jnp.float32),
        compiler_params=pltpu.CompilerParams(needs_layout_passes=False),
        scratch_types=[
            pltpu.VMEM((_N,), jnp.int32),
            pltpu.VMEM((_N,), jnp.float32),
            pltpu.VMEM((_N,), jnp.float32),
        ],
    )
    def scatter(idx_hbm, d_hbm, out_hbm, idx_v, d_v, out_v):
        wid = lax.axis_index("c") * 16 + lax.axis_index("s")
        for r in range(rows_per):
            row = wid * rows_per + r
            pltpu.sync_copy(idx_hbm.at[row], idx_v)
            pltpu.sync_copy(d_hbm.at[row], d_v)

            def body(i, carry):
                iv = idx_v[pl.ds(i * 16, 16)]
                dv = d_v[pl.ds(i * 16, 16)]
                plsc.store_scatter(out_v, [iv], dv)
                return carry

            lax.fori_loop(0, _N // 16, body, 0)
            pltpu.sync_copy(out_v, out_hbm.at[row])

    return scatter


_scatter_fn = None


def _get_scatter():
    global _scatter_fn
    if _scatter_fn is None:
        _scatter_fn = _make_scatter()
    return _scatter_fn


@jax.jit
def _run(x_batch, y_batch, n_projections):
    th = _thetas()
    tho = th * (1.0 / n_projections)

    xp, yp = pl.pallas_call(
        _proj_body,
        grid=(_B,),
        in_specs=[
            pl.BlockSpec((1, _N, _D), lambda b: (b, 0, 0)),
            pl.BlockSpec((1, _N, _D), lambda b: (b, 0, 0)),
            pl.BlockSpec((_P, _D), lambda b: (0, 0)),
        ],
        out_specs=[
            pl.BlockSpec((1, _P, _N), lambda b: (b, 0, 0)),
            pl.BlockSpec((1, _P, _N), lambda b: (b, 0, 0)),
        ],
        out_shape=[
            jax.ShapeDtypeStruct((_B, _P, _N), jnp.float32),
            jax.ShapeDtypeStruct((_B, _P, _N), jnp.float32),
        ],
    )(x_batch, y_batch, th)

    idx_t, d_t = pl.pallas_call(
        _sort_body,
        out_shape=[
            jax.ShapeDtypeStruct((_W, _N), jnp.int32),
            jax.ShapeDtypeStruct((_W, _N), jnp.float32),
        ],
        scratch_shapes=[
            pltpu.VMEM((_N, 2 * _W), jnp.float32),
            pltpu.VMEM((_N, 2 * _W), jnp.int32),
        ],
    )(xp, yp)

    diff_t = _get_scatter()(idx_t, d_t)

    out = pl.pallas_call(
        _final_body,
        grid=(_B,),
        in_specs=[
            pl.BlockSpec((1, _N, _D), lambda b: (b, 0, 0)),
            pl.BlockSpec((_P, _N), lambda b: (b, 0)),
            pl.BlockSpec((_P, _D), lambda b: (0, 0)),
        ],
        out_specs=pl.BlockSpec((1, _N, _D), lambda b: (b, 0, 0)),
        out_shape=jax.ShapeDtypeStruct((_B, _N, _D), jnp.float32),
    )(x_batch, diff_t, tho)
    return out


def kernel(x_batch, y_batch, eps, n_projections):
    return _run(x_batch, y_batch, n_projections)
